# baseline (speedup 1.0000x reference)
import jax
import jax.numpy as jnp
from jax.experimental import pallas as pl
from jax.experimental.pallas import tpu as pltpu

_P = 7
_SCALE = 0.25
_S = 2
_G = 32
_GT = 64


def _roi_kernel(rois_ref, feat_hbm, out_ref, feat_vmem, sem):
    G, P, S = _G, _P, _S
    N, H, W, C = 4, 56, 56, 256
    R = G * P

    @pl.when(pl.program_id(0) == 0)
    def _fetch():
        cp = pltpu.make_async_copy(feat_hbm, feat_vmem, sem)
        cp.start()
        cp.wait()

    rois = rois_ref[...]
    b = rois[:, 0:1]
    x1 = rois[:, 1:2] * _SCALE
    y1 = rois[:, 2:3] * _SCALE
    x2 = rois[:, 3:4] * _SCALE
    y2 = rois[:, 4:5] * _SCALE
    bin_w = jnp.maximum(x2 - x1, 1.0) * (1.0 / P)
    bin_h = jnp.maximum(y2 - y1, 1.0) * (1.0 / P)

    def wmat(origin, binsz, cols, limit, pos_from_col):
        rowi = jax.lax.broadcasted_iota(jnp.int32, (R, cols), 0)
        pf = (rowi % P).astype(jnp.float32)
        colq = jax.lax.broadcasted_iota(jnp.int32, (R, cols), 1)
        pos = pos_from_col(colq)
        acc = jnp.zeros((R, cols), jnp.float32)
        for s in range(S):
            c = origin + (pf + (s + 0.5) / S) * binsz
            valid = (c >= -1.0) & (c <= float(limit))
            cc = jnp.clip(c, 0.0, float(limit - 1))
            w = jnp.maximum(1.0 - jnp.abs(pos - cc), 0.0)
            acc = acc + jnp.where(valid, w, 0.0)
        return acc * (1.0 / S)

    B = wmat(x1, bin_w, N * W, W, lambda q: (q % W).astype(jnp.float32))
    colq = jax.lax.broadcasted_iota(jnp.int32, (R, N * W), 1)
    B = jnp.where((colq // W).astype(jnp.float32) == b, B, 0.0)

    tmp = jax.lax.dot(B, feat_vmem[...], preferred_element_type=jnp.float32)

    A = wmat(y1, bin_h, H, H, lambda q: q.astype(jnp.float32))

    tmp4 = tmp.reshape(G, P, H, C)
    Ay = A.reshape(G, P, H)
    out = jax.lax.dot_general(
        Ay, tmp4,
        dimension_numbers=(((2,), (2,)), ((0,), (0,))),
        preferred_element_type=jnp.float32,
    )
    out_ref[...] = out


def _tr_kernel(x_ref, o_ref):
    o_ref[...] = jnp.swapaxes(x_ref[...], 1, 2)


def kernel(input, rois):
    N, C, H, W = input.shape
    K = rois.shape[0]
    P, G = _P, _G

    feat_x = input.transpose(0, 3, 2, 1).reshape(N * W, H * C)
    rois_rep = jnp.repeat(rois, P, axis=0)

    out = pl.pallas_call(
        _roi_kernel,
        grid=((K + G - 1) // G,),
        in_specs=[
            pl.BlockSpec((G * P, 5), lambda i: (i, 0)),
            pl.BlockSpec(memory_space=pltpu.HBM),
        ],
        out_specs=pl.BlockSpec((G, P, P, C), lambda i: (i, 0, 0, 0)),
        out_shape=jax.ShapeDtypeStruct((K, P, P, C), jnp.float32),
        scratch_shapes=[
            pltpu.VMEM((N * W, H * C), jnp.float32),
            pltpu.SemaphoreType.DMA,
        ],
    )(rois_rep, feat_x)

    out2 = pl.pallas_call(
        _tr_kernel,
        grid=((K + _GT - 1) // _GT,),
        in_specs=[pl.BlockSpec((_GT, P * P, C), lambda i: (i, 0, 0))],
        out_specs=pl.BlockSpec((_GT, C, P * P), lambda i: (i, 0, 0)),
        out_shape=jax.ShapeDtypeStruct((K, C, P * P), jnp.float32),
    )(out.reshape(K, P * P, C))

    return out2.reshape(K, C, P, P)


# speedup vs baseline: 1.0949x; 1.0949x over previous
import jax
import jax.numpy as jnp
from jax.experimental import pallas as pl
from jax.experimental.pallas import tpu as pltpu

_P = 7
_SCALE = 0.25
_S = 2
_G = 32
_GT = 64


def _roi_kernel(rois_ref, feat_hbm, out_ref, feat_vmem, sem):
    G, P, S = _G, _P, _S
    N, H, W, C = 4, 56, 56, 256
    R = G * P

    @pl.when(pl.program_id(0) == 0)
    def _fetch():
        cp = pltpu.make_async_copy(feat_hbm, feat_vmem, sem)
        cp.start()
        cp.wait()

    ri = jax.lax.broadcasted_iota(jnp.int32, (R, G), 0) // P
    gi = jax.lax.broadcasted_iota(jnp.int32, (R, G), 1)
    sel = (ri == gi).astype(jnp.float32)
    rois = jax.lax.dot(sel, rois_ref[...], preferred_element_type=jnp.float32)

    b = rois[:, 0:1]
    x1 = rois[:, 1:2] * _SCALE
    y1 = rois[:, 2:3] * _SCALE
    x2 = rois[:, 3:4] * _SCALE
    y2 = rois[:, 4:5] * _SCALE
    bin_w = jnp.maximum(x2 - x1, 1.0) * (1.0 / P)
    bin_h = jnp.maximum(y2 - y1, 1.0) * (1.0 / P)

    def wmat(origin, binsz, cols, limit, pos_from_col):
        rowi = jax.lax.broadcasted_iota(jnp.int32, (R, cols), 0)
        pf = (rowi % P).astype(jnp.float32)
        colq = jax.lax.broadcasted_iota(jnp.int32, (R, cols), 1)
        pos = pos_from_col(colq)
        acc = jnp.zeros((R, cols), jnp.float32)
        for s in range(S):
            c = origin + (pf + (s + 0.5) / S) * binsz
            valid = (c >= -1.0) & (c <= float(limit))
            cc = jnp.clip(c, 0.0, float(limit - 1))
            w = jnp.maximum(1.0 - jnp.abs(pos - cc), 0.0)
            acc = acc + jnp.where(valid, w, 0.0)
        return acc * (1.0 / S)

    B = wmat(x1, bin_w, N * W, W, lambda q: (q % W).astype(jnp.float32))
    colq = jax.lax.broadcasted_iota(jnp.int32, (R, N * W), 1)
    B = jnp.where((colq // W).astype(jnp.float32) == b, B, 0.0)

    tmp = jax.lax.dot(B, feat_vmem[...], preferred_element_type=jnp.float32)

    A = wmat(y1, bin_h, H, H, lambda q: q.astype(jnp.float32))

    tmp4 = tmp.reshape(G, P, H, C)
    Ay = A.reshape(G, P, H)
    out = jax.lax.dot_general(
        Ay, tmp4,
        dimension_numbers=(((2,), (2,)), ((0,), (0,))),
        preferred_element_type=jnp.float32,
    )
    out_ref[...] = out.reshape(G, P * P, C)


def _tr_kernel(x_ref, o_ref):
    o_ref[...] = jnp.swapaxes(x_ref[...], 1, 2)


def kernel(input, rois):
    N, C, H, W = input.shape
    K = rois.shape[0]
    P, G = _P, _G

    feat_x = input.transpose(0, 3, 2, 1).reshape(N * W, H * C)

    out = pl.pallas_call(
        _roi_kernel,
        grid=((K + G - 1) // G,),
        in_specs=[
            pl.BlockSpec((G, 5), lambda i: (i, 0)),
            pl.BlockSpec(memory_space=pltpu.HBM),
        ],
        out_specs=pl.BlockSpec((G, P * P, C), lambda i: (i, 0, 0)),
        out_shape=jax.ShapeDtypeStruct((K, P * P, C), jnp.float32),
        scratch_shapes=[
            pltpu.VMEM((N * W, H * C), jnp.float32),
            pltpu.SemaphoreType.DMA,
        ],
    )(rois, feat_x)

    out2 = pl.pallas_call(
        _tr_kernel,
        grid=((K + _GT - 1) // _GT,),
        in_specs=[pl.BlockSpec((_GT, P * P, C), lambda i: (i, 0, 0))],
        out_specs=pl.BlockSpec((_GT, C, P * P), lambda i: (i, 0, 0)),
        out_shape=jax.ShapeDtypeStruct((K, C, P * P), jnp.float32),
    )(out)

    return out2.reshape(K, C, P, P)


# speedup vs baseline: 1.0951x; 1.0002x over previous
import jax
import jax.numpy as jnp
from jax.experimental import pallas as pl
from jax.experimental.pallas import tpu as pltpu

_P = 7
_SCALE = 0.25
_S = 2
_G = 32
_GT = 64


def _roi_kernel(rois_ref, feat_hbm, out_ref, feat_vmem, sem):
    G, P, S = _G, _P, _S
    N, H, W, C = 4, 56, 56, 256
    R = G * P

    @pl.when(pl.program_id(0) == 0)
    def _fetch():
        cp = pltpu.make_async_copy(feat_hbm, feat_vmem, sem)
        cp.start()
        cp.wait()

    ri = jax.lax.broadcasted_iota(jnp.int32, (R, G), 0) // P
    gi = jax.lax.broadcasted_iota(jnp.int32, (R, G), 1)
    sel = (ri == gi).astype(jnp.float32)
    rois = jax.lax.dot(sel, rois_ref[...], preferred_element_type=jnp.float32)

    b = rois[:, 0:1]
    x1 = rois[:, 1:2] * _SCALE
    y1 = rois[:, 2:3] * _SCALE
    x2 = rois[:, 3:4] * _SCALE
    y2 = rois[:, 4:5] * _SCALE
    bin_w = jnp.maximum(x2 - x1, 1.0) * (1.0 / P)
    bin_h = jnp.maximum(y2 - y1, 1.0) * (1.0 / P)

    def wmat(origin, binsz, cols, limit, pos_from_col):
        rowi = jax.lax.broadcasted_iota(jnp.int32, (R, cols), 0)
        pf = (rowi % P).astype(jnp.float32)
        colq = jax.lax.broadcasted_iota(jnp.int32, (R, cols), 1)
        pos = pos_from_col(colq)
        acc = jnp.zeros((R, cols), jnp.float32)
        for s in range(S):
            c = origin + (pf + (s + 0.5) / S) * binsz
            valid = (c >= -1.0) & (c <= float(limit))
            cc = jnp.clip(c, 0.0, float(limit - 1))
            w = jnp.maximum(1.0 - jnp.abs(pos - cc), 0.0)
            acc = acc + jnp.where(valid, w, 0.0)
        return acc * (1.0 / S)

    B = wmat(x1, bin_w, N * W, W, lambda q: (q % W).astype(jnp.float32))
    colq = jax.lax.broadcasted_iota(jnp.int32, (R, N * W), 1)
    B = jnp.where((colq // W).astype(jnp.float32) == b, B, 0.0)

    tmp = jax.lax.dot(B, feat_vmem[...], preferred_element_type=jnp.float32)

    A = wmat(y1, bin_h, H, H, lambda q: q.astype(jnp.float32))

    tmp4 = tmp.reshape(G, P, H, C)
    Ay = A.reshape(G, P, H)
    out = jax.lax.dot_general(
        Ay, tmp4,
        dimension_numbers=(((2,), (2,)), ((0,), (0,))),
        preferred_element_type=jnp.float32,
    )
    out_ref[...] = out.reshape(G, P * P, C)


def _tr_kernel(x_ref, o_ref):
    o_ref[...] = jnp.swapaxes(x_ref[...], 1, 2)


def kernel(input, rois):
    N, C, H, W = input.shape
    K = rois.shape[0]
    P, G = _P, _G

    feat_x = input.transpose(0, 3, 2, 1).reshape(N * W, H * C)
    Kp = ((K + G - 1) // G) * G
    rois_p = jnp.zeros((Kp, 5), jnp.float32).at[:K].set(rois)

    out = pl.pallas_call(
        _roi_kernel,
        grid=((K + G - 1) // G,),
        in_specs=[
            pl.BlockSpec((G, 5), lambda i: (i, 0)),
            pl.BlockSpec(memory_space=pltpu.HBM),
        ],
        out_specs=pl.BlockSpec((G, P * P, C), lambda i: (i, 0, 0)),
        out_shape=jax.ShapeDtypeStruct((K, P * P, C), jnp.float32),
        scratch_shapes=[
            pltpu.VMEM((N * W, H * C), jnp.float32),
            pltpu.SemaphoreType.DMA,
        ],
    )(rois_p, feat_x)

    out2 = pl.pallas_call(
        _tr_kernel,
        grid=((K + _GT - 1) // _GT,),
        in_specs=[pl.BlockSpec((_GT, P * P, C), lambda i: (i, 0, 0))],
        out_specs=pl.BlockSpec((_GT, C, P * P), lambda i: (i, 0, 0)),
        out_shape=jax.ShapeDtypeStruct((K, C, P * P), jnp.float32),
    )(out)

    return out2.reshape(K, C, P, P)


# speedup vs baseline: 1.0991x; 1.0036x over previous
import jax
import jax.numpy as jnp
from jax.experimental import pallas as pl
from jax.experimental.pallas import tpu as pltpu

_P = 7
_SCALE = 0.25
_S = 2
_G = 32
_GT = 64


def _roi_kernel(rois_ref, feat_hbm, out_ref, feat_vmem, sem):
    G, P, S = _G, _P, _S
    N, H, W, C = 4, 56, 56, 256
    R = G * P

    @pl.when(pl.program_id(0) == 0)
    def _fetch():
        cp = pltpu.make_async_copy(feat_hbm, feat_vmem, sem)
        cp.start()
        cp.wait()

    rois = rois_ref[...]

    b = rois[:, 0:1]
    x1 = rois[:, 1:2] * _SCALE
    y1 = rois[:, 2:3] * _SCALE
    x2 = rois[:, 3:4] * _SCALE
    y2 = rois[:, 4:5] * _SCALE
    bin_w = jnp.maximum(x2 - x1, 1.0) * (1.0 / P)
    bin_h = jnp.maximum(y2 - y1, 1.0) * (1.0 / P)

    def wmat(origin, binsz, cols, limit, pos_from_col):
        rowi = jax.lax.broadcasted_iota(jnp.int32, (R, cols), 0)
        pf = (rowi % P).astype(jnp.float32)
        colq = jax.lax.broadcasted_iota(jnp.int32, (R, cols), 1)
        pos = pos_from_col(colq)
        acc = jnp.zeros((R, cols), jnp.float32)
        for s in range(S):
            c = origin + (pf + (s + 0.5) / S) * binsz
            valid = (c >= -1.0) & (c <= float(limit))
            cc = jnp.clip(c, 0.0, float(limit - 1))
            w = jnp.maximum(1.0 - jnp.abs(pos - cc), 0.0)
            acc = acc + jnp.where(valid, w, 0.0)
        return acc * (1.0 / S)

    B = wmat(x1, bin_w, N * W, W, lambda q: (q % W).astype(jnp.float32))
    colq = jax.lax.broadcasted_iota(jnp.int32, (R, N * W), 1)
    B = jnp.where((colq // W).astype(jnp.float32) == b, B, 0.0)

    tmp = jax.lax.dot(B, feat_vmem[...], preferred_element_type=jnp.float32)

    A = wmat(y1, bin_h, H, H, lambda q: q.astype(jnp.float32))

    tmp4 = tmp.reshape(G, P, H, C)
    Ay = A.reshape(G, P, H)
    out = jax.lax.dot_general(
        Ay, tmp4,
        dimension_numbers=(((2,), (2,)), ((0,), (0,))),
        preferred_element_type=jnp.float32,
    )
    out_ref[...] = out.reshape(G, P * P, C)


def _tr_kernel(x_ref, o_ref):
    o_ref[...] = jnp.swapaxes(x_ref[...], 1, 2)


def kernel(input, rois):
    N, C, H, W = input.shape
    K = rois.shape[0]
    P, G = _P, _G

    feat_x = input.transpose(0, 3, 2, 1).reshape(N * W, H * C)
    Kp = ((K + G - 1) // G) * G
    rois_p = jnp.zeros((Kp, 5), jnp.float32).at[:K].set(rois)
    rois_rep = jnp.repeat(rois_p, P, axis=0)

    out = pl.pallas_call(
        _roi_kernel,
        grid=((K + G - 1) // G,),
        in_specs=[
            pl.BlockSpec((G * P, 5), lambda i: (i, 0)),
            pl.BlockSpec(memory_space=pltpu.HBM),
        ],
        out_specs=pl.BlockSpec((G, P * P, C), lambda i: (i, 0, 0)),
        out_shape=jax.ShapeDtypeStruct((K, P * P, C), jnp.float32),
        scratch_shapes=[
            pltpu.VMEM((N * W, H * C), jnp.float32),
            pltpu.SemaphoreType.DMA,
        ],
    )(rois_rep, feat_x)

    out2 = pl.pallas_call(
        _tr_kernel,
        grid=((K + _GT - 1) // _GT,),
        in_specs=[pl.BlockSpec((_GT, P * P, C), lambda i: (i, 0, 0))],
        out_specs=pl.BlockSpec((_GT, C, P * P), lambda i: (i, 0, 0)),
        out_shape=jax.ShapeDtypeStruct((K, C, P * P), jnp.float32),
    )(out)

    return out2.reshape(K, C, P, P)


# speedup vs baseline: 1.1138x; 1.0134x over previous
import jax
import jax.numpy as jnp
from jax.experimental import pallas as pl
from jax.experimental.pallas import tpu as pltpu

_P = 7
_SCALE = 0.25
_S = 2
_G = 32
_GT = 64


def _roi_kernel(rois_ref, feat_hbm, out_ref, feat_vmem, sem):
    G, P, S = _G, _P, _S
    N, H, W, C = 4, 56, 56, 256
    R = G * P

    @pl.when(pl.program_id(0) == 0)
    def _fetch():
        cp = pltpu.make_async_copy(feat_hbm, feat_vmem, sem)
        cp.start()
        cp.wait()

    rois = rois_ref[...]

    b = rois[:, 0:1]
    x1 = rois[:, 1:2] * _SCALE
    y1 = rois[:, 2:3] * _SCALE
    x2 = rois[:, 3:4] * _SCALE
    y2 = rois[:, 4:5] * _SCALE
    bin_w = jnp.maximum(x2 - x1, 1.0) * (1.0 / P)
    bin_h = jnp.maximum(y2 - y1, 1.0) * (1.0 / P)

    def wmat(origin, binsz, cols, limit, pos_from_col):
        rowi = jax.lax.broadcasted_iota(jnp.int32, (R, cols), 0)
        pf = (rowi % P).astype(jnp.float32)
        colq = jax.lax.broadcasted_iota(jnp.int32, (R, cols), 1)
        pos = pos_from_col(colq)
        acc = jnp.zeros((R, cols), jnp.float32)
        for s in range(S):
            c = origin + (pf + (s + 0.5) / S) * binsz
            valid = (c >= -1.0) & (c <= float(limit))
            cc = jnp.clip(c, 0.0, float(limit - 1))
            w = jnp.maximum(1.0 - jnp.abs(pos - cc), 0.0)
            acc = acc + jnp.where(valid, w, 0.0)
        return acc * (1.0 / S)

    B = wmat(x1, bin_w, N * W, W, lambda q: (q % W).astype(jnp.float32))
    colq = jax.lax.broadcasted_iota(jnp.int32, (R, N * W), 1)
    B = jnp.where((colq // W).astype(jnp.float32) == b, B, 0.0)

    tmp = jax.lax.dot(B.astype(jnp.bfloat16), feat_vmem[...],
                      preferred_element_type=jnp.float32)

    A = wmat(y1, bin_h, H, H, lambda q: q.astype(jnp.float32))

    tmp4 = tmp.reshape(G, P, H, C)
    Ay = A.reshape(G, P, H)
    out = jax.lax.dot_general(
        Ay, tmp4,
        dimension_numbers=(((2,), (2,)), ((0,), (0,))),
        preferred_element_type=jnp.float32,
    )
    out_ref[...] = out.reshape(G, P * P, C)


def _tr_kernel(x_ref, o_ref):
    o_ref[...] = jnp.swapaxes(x_ref[...], 1, 2)


def kernel(input, rois):
    N, C, H, W = input.shape
    K = rois.shape[0]
    P, G = _P, _G

    feat_x = input.transpose(0, 3, 2, 1).reshape(N * W, H * C).astype(jnp.bfloat16)
    Kp = ((K + G - 1) // G) * G
    rois_p = jnp.zeros((Kp, 5), jnp.float32).at[:K].set(rois)
    rois_rep = jnp.repeat(rois_p, P, axis=0)

    out = pl.pallas_call(
        _roi_kernel,
        grid=((K + G - 1) // G,),
        in_specs=[
            pl.BlockSpec((G * P, 5), lambda i: (i, 0)),
            pl.BlockSpec(memory_space=pltpu.HBM),
        ],
        out_specs=pl.BlockSpec((G, P * P, C), lambda i: (i, 0, 0)),
        out_shape=jax.ShapeDtypeStruct((K, P * P, C), jnp.float32),
        scratch_shapes=[
            pltpu.VMEM((N * W, H * C), jnp.bfloat16),
            pltpu.SemaphoreType.DMA,
        ],
    )(rois_rep, feat_x)

    out2 = pl.pallas_call(
        _tr_kernel,
        grid=((K + _GT - 1) // _GT,),
        in_specs=[pl.BlockSpec((_GT, P * P, C), lambda i: (i, 0, 0))],
        out_specs=pl.BlockSpec((_GT, C, P * P), lambda i: (i, 0, 0)),
        out_shape=jax.ShapeDtypeStruct((K, C, P * P), jnp.float32),
    )(out)

    return out2.reshape(K, C, P, P)


# speedup vs baseline: 1.1583x; 1.0399x over previous
import jax
import jax.numpy as jnp
from jax.experimental import pallas as pl
from jax.experimental.pallas import tpu as pltpu

_P = 7
_SCALE = 0.25
_S = 2
_G = 32
_GT = 64


def _roi_kernel(rois_ref, feat_hbm, out_ref, feat_vmem, sem):
    G, P, S = _G, _P, _S
    N, H, W, C = 4, 56, 56, 256
    R = G * P

    @pl.when(pl.program_id(0) == 0)
    def _fetch():
        cp = pltpu.make_async_copy(feat_hbm, feat_vmem, sem)
        cp.start()
        cp.wait()

    rois = rois_ref[...]

    b = rois[:, 0:1]
    x1 = rois[:, 1:2] * _SCALE
    y1 = rois[:, 2:3] * _SCALE
    x2 = rois[:, 3:4] * _SCALE
    y2 = rois[:, 4:5] * _SCALE
    bin_w = jnp.maximum(x2 - x1, 1.0) * (1.0 / P)
    bin_h = jnp.maximum(y2 - y1, 1.0) * (1.0 / P)

    def wmat(origin, binsz, cols, limit, pos_from_col):
        rowi = jax.lax.broadcasted_iota(jnp.int32, (R, cols), 0)
        pf = (rowi % P).astype(jnp.float32)
        colq = jax.lax.broadcasted_iota(jnp.int32, (R, cols), 1)
        pos = pos_from_col(colq)
        acc = jnp.zeros((R, cols), jnp.float32)
        for s in range(S):
            c = origin + (pf + (s + 0.5) / S) * binsz
            valid = (c >= -1.0) & (c <= float(limit))
            cc = jnp.clip(c, 0.0, float(limit - 1))
            w = jnp.maximum(1.0 - jnp.abs(pos - cc), 0.0)
            acc = acc + jnp.where(valid, w, 0.0)
        return acc * (1.0 / S)

    B = wmat(x1, bin_w, N * W, W, lambda q: (q % W).astype(jnp.float32))
    colq = jax.lax.broadcasted_iota(jnp.int32, (R, N * W), 1)
    B = jnp.where((colq // W).astype(jnp.float32) == b, B, 0.0)

    tmp = jax.lax.dot(B.astype(jnp.bfloat16), feat_vmem[...],
                      preferred_element_type=jnp.float32).astype(jnp.bfloat16)

    A = wmat(y1, bin_h, H, H, lambda q: q.astype(jnp.float32))

    tmp4 = tmp.reshape(G, P, H, C)
    Ay = A.reshape(G, P, H)
    out = jax.lax.dot_general(
        Ay.astype(jnp.bfloat16), tmp4,
        dimension_numbers=(((2,), (2,)), ((0,), (0,))),
        preferred_element_type=jnp.float32,
    )
    out_ref[...] = out.reshape(G, P * P, C)


def _tr_kernel(x_ref, o_ref):
    o_ref[...] = jnp.swapaxes(x_ref[...], 1, 2)


def kernel(input, rois):
    N, C, H, W = input.shape
    K = rois.shape[0]
    P, G = _P, _G

    feat_x = input.transpose(0, 3, 2, 1).reshape(N * W, H * C).astype(jnp.bfloat16)
    Kp = ((K + G - 1) // G) * G
    rois_p = jnp.zeros((Kp, 5), jnp.float32).at[:K].set(rois)
    rois_rep = jnp.repeat(rois_p, P, axis=0)

    out = pl.pallas_call(
        _roi_kernel,
        grid=((K + G - 1) // G,),
        in_specs=[
            pl.BlockSpec((G * P, 5), lambda i: (i, 0)),
            pl.BlockSpec(memory_space=pltpu.HBM),
        ],
        out_specs=pl.BlockSpec((G, P * P, C), lambda i: (i, 0, 0)),
        out_shape=jax.ShapeDtypeStruct((K, P * P, C), jnp.float32),
        scratch_shapes=[
            pltpu.VMEM((N * W, H * C), jnp.bfloat16),
            pltpu.SemaphoreType.DMA,
        ],
    )(rois_rep, feat_x)

    out2 = pl.pallas_call(
        _tr_kernel,
        grid=((K + _GT - 1) // _GT,),
        in_specs=[pl.BlockSpec((_GT, P * P, C), lambda i: (i, 0, 0))],
        out_specs=pl.BlockSpec((_GT, C, P * P), lambda i: (i, 0, 0)),
        out_shape=jax.ShapeDtypeStruct((K, C, P * P), jnp.float32),
    )(out)

    return out2.reshape(K, C, P, P)
